# Initial kernel scaffold; baseline (speedup 1.0000x reference)
import jax
import jax.numpy as jnp
from jax import lax
from jax.experimental import pallas as pl
from jax.experimental.pallas import tpu as pltpu
from jax.experimental.pallas import tpu_sc as plsc

DIM = 64
SEQ = 77
BATCH = 16384

NC = 2
NS = 16
NW = NC * NS
LANES = 16

CB = 8
ROWS = CB * SEQ
GSUB = 88
NG = ROWS // GSUB
B_PER_W = BATCH // NW
CHUNKS = B_PER_W // CB
NSTEP = CHUNKS // 2


def _sc_body(x_hbm, table_hbm, pos_hbm, out_hbm,
             pos_v, idx0, idx1, buf0, buf1, gather_sem, out_sem, idx_sem):
    wid = lax.axis_index("s") * NC + lax.axis_index("c")
    base_row = wid * (B_PER_W * SEQ)

    pltpu.sync_copy(pos_hbm, pos_v)

    def fetch_idx(c, idx_v):
        row0 = base_row + c * ROWS
        return pltpu.async_copy(x_hbm.at[pl.ds(row0, ROWS)], idx_v, idx_sem)

    def wait_idx(idx_v):
        pltpu.make_async_copy(x_hbm.at[pl.ds(0, ROWS)], idx_v, idx_sem).wait()

    def start_gathers(idx_v, buf_v):
        for j in range(NG):
            sl = pl.ds(j * GSUB, GSUB)
            pltpu.async_copy(table_hbm.at[idx_v.at[sl]], buf_v.at[sl],
                             gather_sem)

    def wait_gathers(buf_v):
        pltpu.make_async_copy(table_hbm.at[pl.ds(0, ROWS)], buf_v,
                              gather_sem).wait()

    def wait_out(buf_v):
        pltpu.make_async_copy(buf_v, out_hbm.at[pl.ds(base_row, ROWS)],
                              out_sem).wait()

    def add_pos(buf_v):
        def body(s, _):
            for j in range(DIM // LANES):
                sl = pl.ds(j * LANES, LANES)
                p = pos_v[s, sl]
                for bb in range(CB):
                    r = bb * SEQ + s
                    buf_v[r, sl] = buf_v[r, sl] + p
            return 0
        lax.fori_loop(0, SEQ, body, 0)

    def write_out(c, buf_v):
        row0 = base_row + c * ROWS
        pltpu.async_copy(buf_v, out_hbm.at[pl.ds(row0, ROWS)], out_sem)

    fetch_idx(0, idx0).wait()
    start_gathers(idx0, buf0)
    fetch_idx(1, idx1)

    def step_body(step, _):
        c0 = step * 2
        c1 = c0 + 1

        wait_gathers(buf0)
        wait_idx(idx1)

        @pl.when(step >= 1)
        def _():
            wait_out(buf1)
        start_gathers(idx1, buf1)

        @pl.when(step < NSTEP - 1)
        def _():
            fetch_idx(c0 + 2, idx0)
        add_pos(buf0)
        write_out(c0, buf0)

        wait_gathers(buf1)

        @pl.when(step < NSTEP - 1)
        def _():
            wait_idx(idx0)
            wait_out(buf0)
            start_gathers(idx0, buf0)

        @pl.when(step < NSTEP - 1)
        def _():
            fetch_idx(c1 + 2, idx1)
        add_pos(buf1)
        write_out(c1, buf1)
        return 0

    lax.fori_loop(0, NSTEP, step_body, 0)

    wait_out(buf0)
    wait_out(buf1)


@jax.jit
def kernel(x, token_table, position_embedding):
    batch, seq = x.shape
    x_flat = x.reshape(batch * seq).astype(jnp.int32)
    pos = position_embedding.reshape(position_embedding.shape[1], DIM)

    run = pl.kernel(
        _sc_body,
        out_type=jax.ShapeDtypeStruct((batch * seq, DIM), jnp.float32),
        mesh=plsc.VectorSubcoreMesh(
            core_axis_name="c", subcore_axis_name="s",
            num_cores=NC, num_subcores=NS),
        scratch_types=[
            pltpu.VMEM((seq, DIM), jnp.float32),
            pltpu.VMEM((ROWS,), jnp.int32),
            pltpu.VMEM((ROWS,), jnp.int32),
            pltpu.VMEM((ROWS, DIM), jnp.float32),
            pltpu.VMEM((ROWS, DIM), jnp.float32),
            pltpu.SemaphoreType.DMA,
            pltpu.SemaphoreType.DMA,
            pltpu.SemaphoreType.DMA,
        ],
    )
    out_flat = run(x_flat, token_table, pos)
    return out_flat.reshape(batch, seq, DIM)


# speedup vs baseline: 5.3812x; 5.3812x over previous
import jax
import jax.numpy as jnp
from jax import lax
from jax.experimental import pallas as pl
from jax.experimental.pallas import tpu as pltpu
from jax.experimental.pallas import tpu_sc as plsc

DIM = 64
SEQ = 77
BATCH = 16384

NC = 2
NS = 16
NW = NC * NS
LANES = 16

CB = 8
ROWS = CB * SEQ
GSUB = 88
NG = ROWS // GSUB
B_PER_W = BATCH // NW
CHUNKS = B_PER_W // CB
NSTEP = CHUNKS // 2


def _sc_body(x_hbm, table_hbm, pos_hbm, out_hbm,
             pos_v, idx0, idx1, buf0, buf1, gather_sem, out_sem, idx_sem):
    wid = lax.axis_index("s") * NC + lax.axis_index("c")
    base_row = wid * (B_PER_W * SEQ)

    pltpu.sync_copy(pos_hbm, pos_v)

    def fetch_idx(c, idx_v):
        row0 = base_row + c * ROWS
        return pltpu.async_copy(x_hbm.at[pl.ds(row0, ROWS)], idx_v, idx_sem)

    def wait_idx(idx_v):
        pltpu.make_async_copy(x_hbm.at[pl.ds(0, ROWS)], idx_v, idx_sem).wait()

    def start_gathers(idx_v, buf_v):
        for j in range(NG):
            sl = pl.ds(j * GSUB, GSUB)
            pltpu.async_copy(table_hbm.at[idx_v.at[sl]], buf_v.at[sl],
                             gather_sem)

    def wait_gathers(buf_v):
        pltpu.make_async_copy(table_hbm.at[pl.ds(0, ROWS)], buf_v,
                              gather_sem).wait()

    def wait_out(buf_v):
        pltpu.make_async_copy(buf_v, out_hbm.at[pl.ds(base_row, ROWS)],
                              out_sem).wait()

    def add_pos(buf_v):
        def body(s, _):
            for j in range(DIM // LANES):
                sl = pl.ds(j * LANES, LANES)
                p = pos_v[s, sl]
                for bb in range(CB):
                    r = bb * SEQ + s
                    buf_v[r, sl] = buf_v[r, sl] + p
            return 0
        lax.fori_loop(0, SEQ, body, 0)

    def write_out(c, buf_v):
        row0 = base_row + c * ROWS
        pltpu.async_copy(buf_v, out_hbm.at[pl.ds(row0, ROWS)], out_sem)

    fetch_idx(0, idx0).wait()
    start_gathers(idx0, buf0)
    fetch_idx(1, idx1)

    def step_body(step, _):
        c0 = step * 2
        c1 = c0 + 1

        wait_gathers(buf0)
        wait_idx(idx1)

        @pl.when(step >= 1)
        def _():
            wait_out(buf1)
        start_gathers(idx1, buf1)

        @pl.when(step < NSTEP - 1)
        def _():
            fetch_idx(c0 + 2, idx0)
        add_pos(buf0)
        write_out(c0, buf0)

        wait_gathers(buf1)

        @pl.when(step < NSTEP - 1)
        def _():
            wait_idx(idx0)
            wait_out(buf0)
            start_gathers(idx0, buf0)

        @pl.when(step < NSTEP - 1)
        def _():
            fetch_idx(c1 + 2, idx1)
        add_pos(buf1)
        write_out(c1, buf1)
        return 0

    lax.fori_loop(0, NSTEP, step_body, 0)

    wait_out(buf0)
    wait_out(buf1)


@jax.jit
def kernel(x, token_table, position_embedding):
    batch, seq = x.shape
    x_flat = x.reshape(batch * seq).astype(jnp.int32)
    pos = position_embedding.reshape(position_embedding.shape[1], DIM)

    run = pl.kernel(
        _sc_body,
        out_type=jax.ShapeDtypeStruct((batch * seq, DIM), jnp.float32),
        mesh=plsc.VectorSubcoreMesh(
            core_axis_name="c", subcore_axis_name="s",
            num_cores=NC, num_subcores=NS),
        compiler_params=pltpu.CompilerParams(use_tc_tiling_on_sc=False),
        scratch_types=[
            pltpu.VMEM((seq, DIM), jnp.float32),
            pltpu.VMEM((ROWS,), jnp.int32),
            pltpu.VMEM((ROWS,), jnp.int32),
            pltpu.VMEM((ROWS, DIM), jnp.float32),
            pltpu.VMEM((ROWS, DIM), jnp.float32),
            pltpu.SemaphoreType.DMA,
            pltpu.SemaphoreType.DMA,
            pltpu.SemaphoreType.DMA,
        ],
    )
    out_flat = run(x_flat, token_table, pos)
    return out_flat.reshape(batch, seq, DIM)
